# Initial kernel scaffold; baseline (speedup 1.0000x reference)
import functools

import jax
import jax.numpy as jnp
from jax import lax
from jax.experimental import pallas as pl
from jax.experimental.pallas import tpu as pltpu
from jax.experimental.pallas import tpu_sc as plsc

F32 = jnp.float32
NB = 16
DIM = 256
KC = 512
KH = 128
TT = 784
TCTX = 768
FW = 16
NC, NS = 2, 16
NW = NC * NS
ZQ_W = (NB * TT) // NW
GI_W = (NB * TCTX) // NW
ZQ_CH = 56
GI_CH = 48


def _conv1_body(xc_ref, w_ref, b_ref, h_ref):
    xc = xc_ref[0]
    patch = jnp.concatenate(
        [xc[0:56, 0:56, :], xc[0:56, 1:57, :],
         xc[1:57, 0:56, :], xc[1:57, 1:57, :]], axis=-1)
    a = patch.reshape(3136, 48)
    h = jnp.dot(a, w_ref[...], preferred_element_type=F32) + b_ref[0][None, :]
    h_ref[0] = jnp.maximum(h, 0.0)


def _conv2_body(hc_ref, w_ref, b_ref, cb_ref, z_ref, idx_ref):
    hc = hc_ref[0]
    w = w_ref[...]
    z = jnp.dot(hc[0:28, 0:28, :].reshape(784, 1024), w[0],
                preferred_element_type=F32)
    z = z + jnp.dot(hc[0:28, 1:29, :].reshape(784, 1024), w[1],
                    preferred_element_type=F32)
    z = z + jnp.dot(hc[1:29, 0:28, :].reshape(784, 1024), w[2],
                    preferred_element_type=F32)
    z = z + jnp.dot(hc[1:29, 1:29, :].reshape(784, 1024), w[3],
                    preferred_element_type=F32)
    z = z + b_ref[0][None, :]
    cb = cb_ref[...]
    zsq = jnp.sum(z * z, axis=-1, keepdims=True)
    csq = jnp.sum(cb * cb, axis=-1)
    s = lax.dot_general(z, cb, (((1,), (1,)), ((), ())),
                        preferred_element_type=F32)
    d2 = zsq - 2.0 * s + csq[None, :]
    m = jnp.min(d2, axis=-1, keepdims=True)
    io = lax.broadcasted_iota(jnp.int32, (784, KC), 1)
    idx = jnp.min(jnp.where(d2 <= m, io, KC), axis=-1)
    z_ref[0] = z
    idx_ref[0, 0] = idx


def _gitab_body(cb_ref, wih_ref, bih_ref, o_ref):
    o_ref[...] = (jnp.dot(cb_ref[...], wih_ref[...], preferred_element_type=F32)
                  + bih_ref[0][None, :])


_sc_mesh = plsc.VectorSubcoreMesh(core_axis_name="c", subcore_axis_name="s")


@functools.partial(
    pl.kernel,
    out_type=(jax.ShapeDtypeStruct((NB * TT, DIM), F32),
              jax.ShapeDtypeStruct((NB * TCTX, 3 * KH), F32)),
    mesh=_sc_mesh,
    scratch_types=[
        pltpu.VMEM((ZQ_W // ZQ_CH, ZQ_CH), jnp.int32),
        pltpu.VMEM((GI_W // GI_CH, GI_CH), jnp.int32),
        pltpu.VMEM((ZQ_CH, DIM), F32),
        pltpu.VMEM((GI_CH, 3 * KH), F32),
        pltpu.SemaphoreType.DMA,
    ],
)
def _sc_gather(cb_hbm, gtab_hbm, idxa_hbm, idxc_hbm, zq_out, gi_out,
               idxa_v, idxc_v, rows_a, rows_c, sem):
    wid = lax.axis_index("s") * NC + lax.axis_index("c")
    pltpu.sync_copy(idxa_hbm.at[wid], idxa_v)
    pltpu.sync_copy(idxc_hbm.at[wid], idxc_v)
    base_a = wid * ZQ_W
    base_c = wid * GI_W

    def body_a(j, carry):
        pltpu.async_copy(cb_hbm.at[idxa_v.at[j]], rows_a, sem).wait()
        pltpu.sync_copy(rows_a, zq_out.at[pl.ds(base_a + j * ZQ_CH, ZQ_CH)])
        return carry

    lax.fori_loop(0, ZQ_W // ZQ_CH, body_a, 0)

    def body_c(j, carry):
        pltpu.async_copy(gtab_hbm.at[idxc_v.at[j]], rows_c, sem).wait()
        pltpu.sync_copy(rows_c, gi_out.at[pl.ds(base_c + j * GI_CH, GI_CH)])
        return carry

    lax.fori_loop(0, GI_W // GI_CH, body_c, 0)


def _gru_body(gi_ref, whh_ref, bhh_ref, h0_ref, wp_ref, tg_ref,
              acc_ref, nce_ref):
    whh = whh_ref[...]
    bhh = bhh_ref[0][None, :]

    def step(t, h):
        gx = gi_ref[t]
        gh = jnp.dot(h, whh, preferred_element_type=F32) + bhh
        r = jax.nn.sigmoid(gx[:, 0:KH] + gh[:, 0:KH])
        zg = jax.nn.sigmoid(gx[:, KH:2 * KH] + gh[:, KH:2 * KH])
        n = jnp.tanh(gx[:, 2 * KH:3 * KH] + r * gh[:, 2 * KH:3 * KH])
        return (1.0 - zg) * n + zg * h

    h = lax.fori_loop(0, TCTX, step, h0_ref[...])

    io = lax.broadcasted_iota(jnp.int32, (NB, NB), 1)
    lab = lax.broadcasted_iota(jnp.int32, (NB, NB), 0)
    eye = io == lab
    nce_sum = F32(0.0)
    acc_sum = F32(0.0)
    for k in range(FW):
        pred = jnp.dot(h, wp_ref[k], preferred_element_type=F32)
        sc = lax.dot_general(pred, tg_ref[k], (((1,), (1,)), ((), ())),
                             preferred_element_type=F32)
        m = jnp.max(sc, axis=-1, keepdims=True)
        lse = m + jnp.log(jnp.sum(jnp.exp(sc - m), axis=-1, keepdims=True))
        diag = jnp.sum(jnp.where(eye, sc, 0.0), axis=-1, keepdims=True)
        nce_sum = nce_sum + jnp.sum(diag - lse)
        am = jnp.min(jnp.where(sc >= m, io, NB), axis=-1, keepdims=True)
        lab1 = lax.broadcasted_iota(jnp.int32, (NB, 1), 0)
        acc_sum = acc_sum + jnp.sum((am == lab1).astype(F32))
    nce_ref[0, 0] = -nce_sum / F32(FW * NB)
    acc_ref[0, 0] = acc_sum / F32(FW * NB)


def kernel(x, hidden, conv1_w, conv1_b, conv2_w, conv2_b, codebook,
           W_ih, W_hh, b_ih, b_hh, W_pred):
    x_nhwc = jnp.transpose(x, (0, 2, 3, 1))
    xpad = jnp.pad(x_nhwc, ((0, 0), (1, 1), (1, 1), (0, 0)))
    xcol = (xpad.reshape(NB, 57, 2, 57, 2, 3)
            .transpose(0, 1, 3, 2, 4, 5).reshape(NB, 57, 57, 12))
    w1 = (conv1_w.transpose(2, 3, 1, 0).reshape(2, 2, 2, 2, 3, DIM)
          .transpose(0, 2, 1, 3, 4, 5).reshape(48, DIM))

    h = pl.pallas_call(
        _conv1_body,
        grid=(NB,),
        in_specs=[
            pl.BlockSpec((1, 57, 57, 12), lambda b: (b, 0, 0, 0)),
            pl.BlockSpec((48, DIM), lambda b: (0, 0)),
            pl.BlockSpec((1, DIM), lambda b: (0, 0)),
        ],
        out_specs=pl.BlockSpec((1, 3136, DIM), lambda b: (b, 0, 0)),
        out_shape=jax.ShapeDtypeStruct((NB, 3136, DIM), F32),
    )(xcol, w1, conv1_b.reshape(1, DIM))

    hpad = jnp.pad(h.reshape(NB, 56, 56, DIM), ((0, 0), (1, 1), (1, 1), (0, 0)))
    hcol = (hpad.reshape(NB, 29, 2, 29, 2, DIM)
            .transpose(0, 1, 3, 2, 4, 5).reshape(NB, 29, 29, 4 * DIM))
    w2 = (conv2_w.transpose(2, 3, 1, 0).reshape(2, 2, 2, 2, DIM, DIM)
          .transpose(0, 2, 1, 3, 4, 5).reshape(4, 4 * DIM, DIM))

    z_e, idx3 = pl.pallas_call(
        _conv2_body,
        grid=(NB,),
        in_specs=[
            pl.BlockSpec((1, 29, 29, 4 * DIM), lambda b: (b, 0, 0, 0)),
            pl.BlockSpec((4, 4 * DIM, DIM), lambda b: (0, 0, 0)),
            pl.BlockSpec((1, DIM), lambda b: (0, 0)),
            pl.BlockSpec((KC, DIM), lambda b: (0, 0)),
        ],
        out_specs=[
            pl.BlockSpec((1, TT, DIM), lambda b: (b, 0, 0)),
            pl.BlockSpec((1, 1, TT), lambda b: (b, 0, 0)),
        ],
        out_shape=[
            jax.ShapeDtypeStruct((NB, TT, DIM), F32),
            jax.ShapeDtypeStruct((NB, 1, TT), jnp.int32),
        ],
    )(hcol, w2, conv2_b.reshape(1, DIM), codebook)

    gi_tab = pl.pallas_call(
        _gitab_body,
        in_specs=[
            pl.BlockSpec((KC, DIM), lambda: (0, 0)),
            pl.BlockSpec((DIM, 3 * KH), lambda: (0, 0)),
            pl.BlockSpec((1, 3 * KH), lambda: (0, 0)),
        ],
        out_specs=pl.BlockSpec((KC, 3 * KH), lambda: (0, 0)),
        out_shape=jax.ShapeDtypeStruct((KC, 3 * KH), F32),
    )(codebook, W_ih.T, b_ih.reshape(1, 3 * KH))

    idx = idx3.reshape(NB, TT)
    idx_all = idx.reshape(NW, ZQ_W // ZQ_CH, ZQ_CH)
    idx_ctx = idx[:, :TCTX].T.reshape(NW, GI_W // GI_CH, GI_CH)

    z_q_flat, gi = _sc_gather(codebook, gi_tab, idx_all, idx_ctx)

    targets = jnp.transpose(z_e[:, TCTX:TCTX + FW, :], (1, 0, 2))

    acc2, nce2 = pl.pallas_call(
        _gru_body,
        in_specs=[
            pl.BlockSpec((TCTX, NB, 3 * KH), lambda: (0, 0, 0)),
            pl.BlockSpec((KH, 3 * KH), lambda: (0, 0)),
            pl.BlockSpec((1, 3 * KH), lambda: (0, 0)),
            pl.BlockSpec((NB, KH), lambda: (0, 0)),
            pl.BlockSpec((FW, KH, DIM), lambda: (0, 0, 0)),
            pl.BlockSpec((FW, NB, DIM), lambda: (0, 0, 0)),
        ],
        out_specs=[
            pl.BlockSpec((1, 1), lambda: (0, 0)),
            pl.BlockSpec((1, 1), lambda: (0, 0)),
        ],
        out_shape=[
            jax.ShapeDtypeStruct((1, 1), F32),
            jax.ShapeDtypeStruct((1, 1), F32),
        ],
    )(gi.reshape(TCTX, NB, 3 * KH), W_hh.T, b_hh.reshape(1, 3 * KH),
      hidden[0], W_pred, targets)

    z_e_x = jnp.transpose(z_e.reshape(NB, 28, 28, DIM), (0, 3, 1, 2))
    z_q_x = jnp.transpose(z_q_flat.reshape(NB, 28, 28, DIM), (0, 3, 1, 2))
    return acc2[0, 0], nce2[0, 0], z_e_x, z_q_x


# speedup vs baseline: 1.3467x; 1.3467x over previous
import functools

import jax
import jax.numpy as jnp
from jax import lax
from jax.experimental import pallas as pl
from jax.experimental.pallas import tpu as pltpu
from jax.experimental.pallas import tpu_sc as plsc

F32 = jnp.float32
NB = 16
DIM = 256
KC = 512
KH = 128
TT = 784
TCTX = 768
FW = 16
NC, NS = 2, 16
NW = NC * NS
ZQ_W = (NB * TT) // NW
GI_W = (NB * TCTX) // NW
ZQ_CH = 56
GI_CH = 48


def _conv1_body(xc_ref, w_ref, b_ref, h_ref):
    xc = xc_ref[0]
    patch = jnp.concatenate(
        [xc[0:56, 0:56, :], xc[0:56, 1:57, :],
         xc[1:57, 0:56, :], xc[1:57, 1:57, :]], axis=-1)
    a = patch.reshape(3136, 48)
    h = jnp.dot(a, w_ref[...], preferred_element_type=F32) + b_ref[0][None, :]
    h_ref[0] = jnp.maximum(h, 0.0)


def _conv2_body(hc_ref, w_ref, b_ref, cb_ref, z_ref, idx_ref):
    hc = hc_ref[0]
    w = w_ref[...]
    z = jnp.dot(hc[0:28, 0:28, :].reshape(784, 1024), w[0],
                preferred_element_type=F32)
    z = z + jnp.dot(hc[0:28, 1:29, :].reshape(784, 1024), w[1],
                    preferred_element_type=F32)
    z = z + jnp.dot(hc[1:29, 0:28, :].reshape(784, 1024), w[2],
                    preferred_element_type=F32)
    z = z + jnp.dot(hc[1:29, 1:29, :].reshape(784, 1024), w[3],
                    preferred_element_type=F32)
    z = z + b_ref[0][None, :]
    cb = cb_ref[...]
    zsq = jnp.sum(z * z, axis=-1, keepdims=True)
    csq = jnp.sum(cb * cb, axis=-1)
    s = lax.dot_general(z, cb, (((1,), (1,)), ((), ())),
                        preferred_element_type=F32)
    d2 = zsq - 2.0 * s + csq[None, :]
    m = jnp.min(d2, axis=-1, keepdims=True)
    io = lax.broadcasted_iota(jnp.int32, (784, KC), 1)
    idx = jnp.min(jnp.where(d2 <= m, io, KC), axis=-1)
    z_ref[0] = z
    idx_ref[0, 0] = idx


def _gitab_body(cb_ref, wih_ref, bih_ref, o_ref):
    o_ref[...] = (jnp.dot(cb_ref[...], wih_ref[...], preferred_element_type=F32)
                  + bih_ref[0][None, :])


@functools.lru_cache(maxsize=1)
def _make_sc_gather():
    mesh = plsc.VectorSubcoreMesh(core_axis_name="c", subcore_axis_name="s")

    @functools.partial(
        pl.kernel,
        out_type=(jax.ShapeDtypeStruct((NB * TT, DIM), F32),
                  jax.ShapeDtypeStruct((NB * TCTX, 3 * KH), F32)),
        mesh=mesh,
        scratch_types=[
            pltpu.VMEM((ZQ_W // ZQ_CH, ZQ_CH), jnp.int32),
            pltpu.VMEM((GI_W // GI_CH, GI_CH), jnp.int32),
            pltpu.VMEM((ZQ_CH, DIM), F32),
            pltpu.VMEM((GI_CH, 3 * KH), F32),
            pltpu.SemaphoreType.DMA,
        ],
    )
    def sc_gather(cb_hbm, gtab_hbm, idxa_hbm, idxc_hbm, zq_out, gi_out,
                  idxa_v, idxc_v, rows_a, rows_c, sem):
        wid = lax.axis_index("s") * NC + lax.axis_index("c")
        pltpu.sync_copy(idxa_hbm.at[wid], idxa_v)
        pltpu.sync_copy(idxc_hbm.at[wid], idxc_v)
        base_a = wid * ZQ_W
        base_c = wid * GI_W

        def body_a(j, carry):
            pltpu.async_copy(cb_hbm.at[idxa_v.at[j]], rows_a, sem).wait()
            pltpu.sync_copy(rows_a, zq_out.at[pl.ds(base_a + j * ZQ_CH, ZQ_CH)])
            return carry

        lax.fori_loop(0, ZQ_W // ZQ_CH, body_a, 0)

        def body_c(j, carry):
            pltpu.async_copy(gtab_hbm.at[idxc_v.at[j]], rows_c, sem).wait()
            pltpu.sync_copy(rows_c, gi_out.at[pl.ds(base_c + j * GI_CH, GI_CH)])
            return carry

        lax.fori_loop(0, GI_W // GI_CH, body_c, 0)

    return sc_gather


def _sc_gather(cb, gtab, idx_all, idx_ctx):
    return _make_sc_gather()(cb, gtab, idx_all, idx_ctx)


def _gru_body(gi_ref, whh_ref, bhh_ref, h0_ref, wp_ref, tg_ref,
              acc_ref, nce_ref):
    whh = whh_ref[...]
    bhh = bhh_ref[0][None, :]

    def step(t, h):
        gx = gi_ref[t]
        gh = jnp.dot(h, whh, preferred_element_type=F32) + bhh
        r = jax.nn.sigmoid(gx[:, 0:KH] + gh[:, 0:KH])
        zg = jax.nn.sigmoid(gx[:, KH:2 * KH] + gh[:, KH:2 * KH])
        n = jnp.tanh(gx[:, 2 * KH:3 * KH] + r * gh[:, 2 * KH:3 * KH])
        return (1.0 - zg) * n + zg * h

    h = lax.fori_loop(0, TCTX, step, h0_ref[...])

    io = lax.broadcasted_iota(jnp.int32, (NB, NB), 1)
    lab = lax.broadcasted_iota(jnp.int32, (NB, NB), 0)
    eye = io == lab
    nce_sum = F32(0.0)
    acc_sum = F32(0.0)
    for k in range(FW):
        pred = jnp.dot(h, wp_ref[k], preferred_element_type=F32)
        sc = lax.dot_general(pred, tg_ref[k], (((1,), (1,)), ((), ())),
                             preferred_element_type=F32)
        m = jnp.max(sc, axis=-1, keepdims=True)
        lse = m + jnp.log(jnp.sum(jnp.exp(sc - m), axis=-1, keepdims=True))
        diag = jnp.sum(jnp.where(eye, sc, 0.0), axis=-1, keepdims=True)
        nce_sum = nce_sum + jnp.sum(diag - lse)
        am = jnp.min(jnp.where(sc >= m, io, NB), axis=-1, keepdims=True)
        lab1 = lax.broadcasted_iota(jnp.int32, (NB, 1), 0)
        acc_sum = acc_sum + jnp.sum((am == lab1).astype(F32))
    nce_ref[...] = jnp.reshape(-nce_sum / F32(FW * NB), (1, 1))
    acc_ref[...] = jnp.reshape(acc_sum / F32(FW * NB), (1, 1))


def kernel(x, hidden, conv1_w, conv1_b, conv2_w, conv2_b, codebook,
           W_ih, W_hh, b_ih, b_hh, W_pred):
    x_nhwc = jnp.transpose(x, (0, 2, 3, 1))
    xpad = jnp.pad(x_nhwc, ((0, 0), (1, 1), (1, 1), (0, 0)))
    xcol = (xpad.reshape(NB, 57, 2, 57, 2, 3)
            .transpose(0, 1, 3, 2, 4, 5).reshape(NB, 57, 57, 12))
    w1 = (conv1_w.transpose(2, 3, 1, 0).reshape(2, 2, 2, 2, 3, DIM)
          .transpose(0, 2, 1, 3, 4, 5).reshape(48, DIM))

    h = pl.pallas_call(
        _conv1_body,
        grid=(NB,),
        in_specs=[
            pl.BlockSpec((1, 57, 57, 12), lambda b: (b, 0, 0, 0)),
            pl.BlockSpec((48, DIM), lambda b: (0, 0)),
            pl.BlockSpec((1, DIM), lambda b: (0, 0)),
        ],
        out_specs=pl.BlockSpec((1, 3136, DIM), lambda b: (b, 0, 0)),
        out_shape=jax.ShapeDtypeStruct((NB, 3136, DIM), F32),
    )(xcol, w1, conv1_b.reshape(1, DIM))

    hpad = jnp.pad(h.reshape(NB, 56, 56, DIM), ((0, 0), (1, 1), (1, 1), (0, 0)))
    hcol = (hpad.reshape(NB, 29, 2, 29, 2, DIM)
            .transpose(0, 1, 3, 2, 4, 5).reshape(NB, 29, 29, 4 * DIM))
    w2 = (conv2_w.transpose(2, 3, 1, 0).reshape(2, 2, 2, 2, DIM, DIM)
          .transpose(0, 2, 1, 3, 4, 5).reshape(4, 4 * DIM, DIM))

    z_e, idx3 = pl.pallas_call(
        _conv2_body,
        grid=(NB,),
        in_specs=[
            pl.BlockSpec((1, 29, 29, 4 * DIM), lambda b: (b, 0, 0, 0)),
            pl.BlockSpec((4, 4 * DIM, DIM), lambda b: (0, 0, 0)),
            pl.BlockSpec((1, DIM), lambda b: (0, 0)),
            pl.BlockSpec((KC, DIM), lambda b: (0, 0)),
        ],
        out_specs=[
            pl.BlockSpec((1, TT, DIM), lambda b: (b, 0, 0)),
            pl.BlockSpec((1, 1, TT), lambda b: (b, 0, 0)),
        ],
        out_shape=[
            jax.ShapeDtypeStruct((NB, TT, DIM), F32),
            jax.ShapeDtypeStruct((NB, 1, TT), jnp.int32),
        ],
    )(hcol, w2, conv2_b.reshape(1, DIM), codebook)

    gi_tab = pl.pallas_call(
        _gitab_body,
        in_specs=[
            pl.BlockSpec((KC, DIM), lambda: (0, 0)),
            pl.BlockSpec((DIM, 3 * KH), lambda: (0, 0)),
            pl.BlockSpec((1, 3 * KH), lambda: (0, 0)),
        ],
        out_specs=pl.BlockSpec((KC, 3 * KH), lambda: (0, 0)),
        out_shape=jax.ShapeDtypeStruct((KC, 3 * KH), F32),
    )(codebook, W_ih.T, b_ih.reshape(1, 3 * KH))

    idx = idx3.reshape(NB, TT)
    idx_all = idx.reshape(NW, ZQ_W // ZQ_CH, ZQ_CH)
    idx_ctx = idx[:, :TCTX].T.reshape(NW, GI_W // GI_CH, GI_CH)

    z_q_flat, gi = _sc_gather(codebook, gi_tab, idx_all, idx_ctx)

    targets = jnp.transpose(z_e[:, TCTX:TCTX + FW, :], (1, 0, 2))

    acc2, nce2 = pl.pallas_call(
        _gru_body,
        in_specs=[
            pl.BlockSpec((TCTX, NB, 3 * KH), lambda: (0, 0, 0)),
            pl.BlockSpec((KH, 3 * KH), lambda: (0, 0)),
            pl.BlockSpec((1, 3 * KH), lambda: (0, 0)),
            pl.BlockSpec((NB, KH), lambda: (0, 0)),
            pl.BlockSpec((FW, KH, DIM), lambda: (0, 0, 0)),
            pl.BlockSpec((FW, NB, DIM), lambda: (0, 0, 0)),
        ],
        out_specs=[
            pl.BlockSpec((1, 1), lambda: (0, 0)),
            pl.BlockSpec((1, 1), lambda: (0, 0)),
        ],
        out_shape=[
            jax.ShapeDtypeStruct((1, 1), F32),
            jax.ShapeDtypeStruct((1, 1), F32),
        ],
    )(gi.reshape(TCTX, NB, 3 * KH), W_hh.T, b_hh.reshape(1, 3 * KH),
      hidden[0], W_pred, targets)

    z_e_x = jnp.transpose(z_e.reshape(NB, 28, 28, DIM), (0, 3, 1, 2))
    z_q_x = jnp.transpose(z_q_flat.reshape(NB, 28, 28, DIM), (0, 3, 1, 2))
    return acc2[0, 0], nce2[0, 0], z_e_x, z_q_x


# speedup vs baseline: 2.5833x; 1.9183x over previous
import functools

import jax
import jax.numpy as jnp
from jax import lax
from jax.experimental import pallas as pl
from jax.experimental.pallas import tpu as pltpu
from jax.experimental.pallas import tpu_sc as plsc

F32 = jnp.float32
NB = 16
DIM = 256
KC = 512
KH = 128
TT = 784
TCTX = 768
FW = 16
NC, NS = 2, 16
NW = NC * NS
ZQ_W = (NB * TT) // NW
GI_W = (NB * TCTX) // NW
GI_CH = 128


def _enc_body(x2_ref, sel_ref, w1_ref, b1_ref, w2_ref, b2_ref, cb_ref,
              zt_ref, idx_ref, tt_ref, hc_ref):
    x2 = x2_ref[0]
    y0 = jnp.dot(sel_ref[0], x2, preferred_element_type=F32)
    y1 = jnp.dot(sel_ref[1], x2, preferred_element_type=F32)
    z0 = y0.reshape(57, 57, 6)
    z1 = y1.reshape(57, 57, 6)
    parts = []
    for dh in (0, 1):
        for dw in (0, 1):
            for zz in (z0, z1):
                parts.append(zz[dh:dh + 56, dw:dw + 56, :])
    a = jnp.concatenate(parts, axis=-1).reshape(3136, 48)
    hm = jnp.maximum(
        jnp.dot(a, w1_ref[...], preferred_element_type=F32) + b1_ref[0][None, :],
        0.0)

    h4 = hm.reshape(28, 2, 56, 256)
    he = h4[:, 0].reshape(28, 28, 2, 256)
    ho = h4[:, 1].reshape(28, 28, 2, 256)
    hc_ref[...] = jnp.zeros((29, 29, 4 * DIM), F32)
    hc_ref[1:29, 1:29, 0:256] = ho[:, :, 1, :]
    hc_ref[1:29, 0:28, 256:512] = ho[:, :, 0, :]
    hc_ref[0:28, 1:29, 512:768] = he[:, :, 1, :]
    hc_ref[0:28, 0:28, 768:1024] = he[:, :, 0, :]
    hc = hc_ref[...]

    w2 = w2_ref[...]
    z = jnp.dot(hc[0:28, 0:28, :].reshape(784, 1024), w2[0],
                preferred_element_type=F32)
    z = z + jnp.dot(hc[0:28, 1:29, :].reshape(784, 1024), w2[1],
                    preferred_element_type=F32)
    z = z + jnp.dot(hc[1:29, 0:28, :].reshape(784, 1024), w2[2],
                    preferred_element_type=F32)
    z = z + jnp.dot(hc[1:29, 1:29, :].reshape(784, 1024), w2[3],
                    preferred_element_type=F32)
    z = z + b2_ref[0][None, :]

    cb = cb_ref[...]
    zsq = jnp.sum(z * z, axis=-1, keepdims=True)
    csq = jnp.sum(cb * cb, axis=-1)
    s = lax.dot_general(z, cb, (((1,), (1,)), ((), ())),
                        preferred_element_type=F32)
    d2 = zsq - 2.0 * s + csq[None, :]
    m = jnp.min(d2, axis=-1, keepdims=True)
    io = lax.broadcasted_iota(jnp.int32, (784, KC), 1)
    idx = jnp.min(jnp.where(d2 <= m, io, KC), axis=-1)
    zt_ref[0] = z.T
    idx_ref[0, 0] = idx
    tt_ref[0] = z[TCTX:TT, :]


def _gitab_body(cb_ref, wih_ref, bih_ref, o_ref):
    o_ref[...] = (jnp.dot(cb_ref[...], wih_ref[...], preferred_element_type=F32)
                  + bih_ref[0][None, :])


@functools.lru_cache(maxsize=1)
def _make_sc_gathers():
    mesh = plsc.VectorSubcoreMesh(core_axis_name="c", subcore_axis_name="s")

    @functools.partial(
        pl.kernel,
        out_type=jax.ShapeDtypeStruct((NB * TCTX, 3 * KH), F32),
        mesh=mesh,
        scratch_types=[
            pltpu.VMEM((GI_W // GI_CH, GI_CH), jnp.int32),
            pltpu.VMEM((GI_CH, 3 * KH), F32),
            pltpu.VMEM((GI_CH, 3 * KH), F32),
            pltpu.SemaphoreType.DMA,
            pltpu.SemaphoreType.DMA,
        ],
    )
    def gi_gather(gtab_hbm, idxc_hbm, gi_out, idxc_v, r0, r1, s0, s1):
        wid = lax.axis_index("s") * NC + lax.axis_index("c")
        pltpu.sync_copy(idxc_hbm.at[wid], idxc_v)
        base = wid * GI_W
        cp0 = pltpu.async_copy(gtab_hbm.at[idxc_v.at[0]], r0, s0)
        cp1 = pltpu.async_copy(gtab_hbm.at[idxc_v.at[1]], r1, s1)
        cp0.wait()
        pltpu.sync_copy(r0, gi_out.at[pl.ds(base, GI_CH)])
        cp2 = pltpu.async_copy(gtab_hbm.at[idxc_v.at[2]], r0, s0)
        cp1.wait()
        pltpu.sync_copy(r1, gi_out.at[pl.ds(base + GI_CH, GI_CH)])
        cp2.wait()
        pltpu.sync_copy(r0, gi_out.at[pl.ds(base + 2 * GI_CH, GI_CH)])

    @functools.partial(
        pl.kernel,
        out_type=jax.ShapeDtypeStruct((NB * TT, DIM), F32),
        mesh=mesh,
        scratch_types=[
            pltpu.VMEM((ZQ_W,), jnp.int32),
            pltpu.VMEM((ZQ_W, DIM), F32),
            pltpu.SemaphoreType.DMA,
        ],
    )
    def zq_gather(cb_hbm, idxa_hbm, zq_out, idxa_v, rows, sem):
        wid = lax.axis_index("s") * NC + lax.axis_index("c")
        pltpu.sync_copy(idxa_hbm.at[wid], idxa_v)
        base = wid * ZQ_W
        pltpu.async_copy(cb_hbm.at[idxa_v], rows, sem).wait()
        pltpu.sync_copy(rows, zq_out.at[pl.ds(base, ZQ_W)])

    return gi_gather, zq_gather


def _gru_body(gi_ref, whh_ref, h0_ref, wp_ref, tt_ref, acc_ref, nce_ref):
    whh = whh_ref[...]

    def step(t, h):
        gx = gi_ref[t]
        gh = jnp.dot(h, whh, preferred_element_type=F32)
        r = jax.nn.sigmoid(gx[:, 0:KH] + gh[:, 0:KH])
        zg = jax.nn.sigmoid(gx[:, KH:2 * KH] + gh[:, KH:2 * KH])
        n = jnp.tanh(gx[:, 2 * KH:3 * KH] + r * gh[:, 2 * KH:3 * KH])
        return (1.0 - zg) * n + zg * h

    h = lax.fori_loop(0, TCTX, step, h0_ref[...])

    tt = tt_ref[...]
    io = lax.broadcasted_iota(jnp.int32, (NB, NB), 1)
    lab = lax.broadcasted_iota(jnp.int32, (NB, NB), 0)
    eye = io == lab
    nce_sum = F32(0.0)
    acc_sum = F32(0.0)
    for k in range(FW):
        pred = jnp.dot(h, wp_ref[k], preferred_element_type=F32)
        tg = tt[:, k, :]
        sc = lax.dot_general(pred, tg, (((1,), (1,)), ((), ())),
                             preferred_element_type=F32)
        m = jnp.max(sc, axis=-1, keepdims=True)
        lse = m + jnp.log(jnp.sum(jnp.exp(sc - m), axis=-1, keepdims=True))
        diag = jnp.sum(jnp.where(eye, sc, 0.0), axis=-1, keepdims=True)
        nce_sum = nce_sum + jnp.sum(diag - lse)
        am = jnp.min(jnp.where(sc >= m, io, NB), axis=-1, keepdims=True)
        lab1 = lax.broadcasted_iota(jnp.int32, (NB, 1), 0)
        acc_sum = acc_sum + jnp.sum((am == lab1).astype(F32))
    nce_ref[...] = jnp.reshape(-nce_sum / F32(FW * NB), (1, 1))
    acc_ref[...] = jnp.reshape(acc_sum / F32(FW * NB), (1, 1))


def kernel(x, hidden, conv1_w, conv1_b, conv2_w, conv2_b, codebook,
           W_ih, W_hh, b_ih, b_hh, W_pred):
    x_nhwc = jnp.transpose(x, (0, 2, 3, 1))
    xpad = jnp.pad(x_nhwc, ((0, 0), (1, 1), (1, 1), (0, 0)))
    x2 = xpad.reshape(NB, 114, 342)
    w1 = (conv1_w.transpose(2, 3, 1, 0).reshape(2, 2, 2, 2, 3, DIM)
          .transpose(0, 2, 1, 3, 4, 5).reshape(48, DIM))
    w2 = (conv2_w.transpose(2, 3, 1, 0).reshape(2, 2, 2, 2, DIM, DIM)
          .transpose(0, 2, 1, 3, 4, 5).reshape(4, 4 * DIM, DIM))
    hp = jnp.arange(57)
    rows = jnp.arange(114)
    sel = jnp.stack([
        (rows[None, :] == 2 * hp[:, None]).astype(F32),
        (rows[None, :] == 2 * hp[:, None] + 1).astype(F32)])

    zt, idx3, tt = pl.pallas_call(
        _enc_body,
        grid=(NB,),
        in_specs=[
            pl.BlockSpec((1, 114, 342), lambda b: (b, 0, 0)),
            pl.BlockSpec((2, 57, 114), lambda b: (0, 0, 0)),
            pl.BlockSpec((48, DIM), lambda b: (0, 0)),
            pl.BlockSpec((1, DIM), lambda b: (0, 0)),
            pl.BlockSpec((4, 4 * DIM, DIM), lambda b: (0, 0, 0)),
            pl.BlockSpec((1, DIM), lambda b: (0, 0)),
            pl.BlockSpec((KC, DIM), lambda b: (0, 0)),
        ],
        out_specs=[
            pl.BlockSpec((1, DIM, TT), lambda b: (b, 0, 0)),
            pl.BlockSpec((1, 1, TT), lambda b: (b, 0, 0)),
            pl.BlockSpec((1, FW, DIM), lambda b: (b, 0, 0)),
        ],
        out_shape=[
            jax.ShapeDtypeStruct((NB, DIM, TT), F32),
            jax.ShapeDtypeStruct((NB, 1, TT), jnp.int32),
            jax.ShapeDtypeStruct((NB, FW, DIM), F32),
        ],
        scratch_shapes=[pltpu.VMEM((29, 29, 4 * DIM), F32)],
    )(x2, sel, w1, conv1_b.reshape(1, DIM), w2, conv2_b.reshape(1, DIM),
      codebook)

    gi_tab = pl.pallas_call(
        _gitab_body,
        in_specs=[
            pl.BlockSpec((KC, DIM), lambda: (0, 0)),
            pl.BlockSpec((DIM, 3 * KH), lambda: (0, 0)),
            pl.BlockSpec((1, 3 * KH), lambda: (0, 0)),
        ],
        out_specs=pl.BlockSpec((KC, 3 * KH), lambda: (0, 0)),
        out_shape=jax.ShapeDtypeStruct((KC, 3 * KH), F32),
    )(codebook, W_ih.T, (b_ih + b_hh).reshape(1, 3 * KH))

    idx = idx3.reshape(NB, TT)
    idx_all = idx.reshape(NW, ZQ_W)
    idx_ctx = idx[:, :TCTX].T.reshape(NW, GI_W // GI_CH, GI_CH)

    gi_gather, zq_gather = _make_sc_gathers()
    gi = gi_gather(gi_tab, idx_ctx)
    z_q_flat = zq_gather(codebook, idx_all)

    acc2, nce2 = pl.pallas_call(
        _gru_body,
        in_specs=[
            pl.BlockSpec((TCTX, NB, 3 * KH), lambda: (0, 0, 0)),
            pl.BlockSpec((KH, 3 * KH), lambda: (0, 0)),
            pl.BlockSpec((NB, KH), lambda: (0, 0)),
            pl.BlockSpec((FW, KH, DIM), lambda: (0, 0, 0)),
            pl.BlockSpec((NB, FW, DIM), lambda: (0, 0, 0)),
        ],
        out_specs=[
            pl.BlockSpec((1, 1), lambda: (0, 0)),
            pl.BlockSpec((1, 1), lambda: (0, 0)),
        ],
        out_shape=[
            jax.ShapeDtypeStruct((1, 1), F32),
            jax.ShapeDtypeStruct((1, 1), F32),
        ],
    )(gi.reshape(TCTX, NB, 3 * KH), W_hh.T, hidden[0], W_pred, tt)

    z_e_x = zt.reshape(NB, DIM, 28, 28)
    z_q_x = jnp.transpose(z_q_flat.reshape(NB, 28, 28, DIM), (0, 3, 1, 2))
    return acc2[0, 0], nce2[0, 0], z_e_x, z_q_x
